# baseline (speedup 1.0000x reference)
import functools

import jax
import jax.numpy as jnp
from jax import lax
from jax.experimental import pallas as pl
from jax.experimental.pallas import tpu as pltpu
from jax.experimental.pallas import tpu_sc as plsc


@functools.lru_cache(maxsize=None)
def _make_gather(B, D, NC, NS):
    NW = NC * NS
    assert B % (8 * NW) == 0
    b_per_w = B // NW
    L = 16
    mesh = plsc.VectorSubcoreMesh(core_axis_name="c", subcore_axis_name="s")

    @functools.partial(
        pl.kernel,
        mesh=mesh,
        out_type=jax.ShapeDtypeStruct((D, B), jnp.float32),
        scratch_types=[
            pltpu.VMEM((b_per_w,), jnp.int32),
            pltpu.VMEM((b_per_w, D), jnp.float32),
            pltpu.VMEM((D, b_per_w), jnp.float32),
            pltpu.SemaphoreType.DMA,
        ],
        compiler_params=pltpu.CompilerParams(needs_layout_passes=False),
    )
    def k(table_hbm, idx_hbm, out_t_hbm, idx_v, rows_v, rows_t, sem):
        wid = lax.axis_index("s") * NC + lax.axis_index("c")
        base = wid * b_per_w
        pltpu.sync_copy(idx_hbm.at[pl.ds(base, b_per_w)], idx_v)

        def issue(j, _):
            v = idx_v[pl.ds(j * L, L)]
            for t in range(L):
                pltpu.async_copy(
                    table_hbm.at[pl.ds(v[t], 1)],
                    rows_v.at[pl.ds(j * L + t, 1)],
                    sem,
                )
            return _

        lax.fori_loop(0, b_per_w // L, issue, None)
        pltpu.make_async_copy(
            table_hbm.at[pl.ds(0, b_per_w)], rows_v, sem
        ).wait()

        def transpose(j, _):
            row = lax.iota(jnp.int32, L) + j * L
            for c in range(D):
                cc = jnp.full((L,), c, jnp.int32)
                val = plsc.load_gather(rows_v, [row, cc])
                plsc.store_scatter(rows_t, [cc, row], val)
            return _

        lax.fori_loop(0, b_per_w // L, transpose, None)
        pltpu.sync_copy(rows_t, out_t_hbm.at[:, pl.ds(base, b_per_w)])

    return k


def kernel(style_id, embed_weight):
    V, D = embed_weight.shape
    (B,) = style_id.shape
    info = plsc.get_sparse_core_info()
    idx = style_id.astype(jnp.int32)
    out_t = _make_gather(B, D, info.num_cores, info.num_subcores)(
        embed_weight, idx
    )
    return out_t.T


# speedup vs baseline: 1.1466x; 1.1466x over previous
import functools

import jax
import jax.numpy as jnp
from jax import lax
from jax.experimental import pallas as pl
from jax.experimental.pallas import tpu as pltpu
from jax.experimental.pallas import tpu_sc as plsc


@functools.lru_cache(maxsize=None)
def _make_gather(B, D, NC, NS):
    NW = NC * NS
    assert B % (8 * NW) == 0
    b_per_w = B // NW
    L = 16
    mesh = plsc.VectorSubcoreMesh(core_axis_name="c", subcore_axis_name="s")

    @functools.partial(
        pl.kernel,
        mesh=mesh,
        out_type=jax.ShapeDtypeStruct((D, B), jnp.float32),
        scratch_types=[
            pltpu.VMEM((b_per_w,), jnp.int32),
            pltpu.VMEM((b_per_w, D), jnp.float32),
            pltpu.VMEM((D, b_per_w), jnp.float32),
            pltpu.SemaphoreType.DMA,
        ],
        compiler_params=pltpu.CompilerParams(needs_layout_passes=False),
    )
    def k(table_hbm, idx_hbm, out_t_hbm, idx_v, rows_v, rows_t, sem):
        wid = lax.axis_index("s") * NC + lax.axis_index("c")
        base = wid * b_per_w
        pltpu.sync_copy(idx_hbm.at[pl.ds(base, b_per_w)], idx_v)

        @plsc.parallel_loop(0, b_per_w // L)
        def issue(j):
            v = idx_v[pl.ds(j * L, L)]
            for t in range(L):
                pltpu.async_copy(
                    table_hbm.at[pl.ds(v[t], 1)],
                    rows_v.at[pl.ds(j * L + t, 1)],
                    sem,
                )
        pltpu.make_async_copy(
            table_hbm.at[pl.ds(0, b_per_w)], rows_v, sem
        ).wait()

        @plsc.parallel_loop(0, b_per_w // L)
        def transpose(j):
            row = lax.iota(jnp.int32, L) + j * L
            for c in range(D):
                cc = jnp.full((L,), c, jnp.int32)
                val = plsc.load_gather(rows_v, [row, cc])
                plsc.store_scatter(rows_t, [cc, row], val)
        pltpu.sync_copy(rows_t, out_t_hbm.at[:, pl.ds(base, b_per_w)])

    return k


def kernel(style_id, embed_weight):
    V, D = embed_weight.shape
    (B,) = style_id.shape
    info = plsc.get_sparse_core_info()
    idx = style_id.astype(jnp.int32)
    out_t = _make_gather(B, D, info.num_cores, info.num_subcores)(
        embed_weight, idx
    )
    return out_t.T


# speedup vs baseline: 1.2360x; 1.0779x over previous
import functools

import jax
import jax.numpy as jnp
from jax import lax
from jax.experimental import pallas as pl
from jax.experimental.pallas import tpu as pltpu
from jax.experimental.pallas import tpu_sc as plsc

_NCHUNK = 4


@functools.lru_cache(maxsize=None)
def _make_gather(B, D, NC, NS):
    NW = NC * NS
    assert B % (8 * NW) == 0
    b_per_w = B // NW
    L = 16
    CH = b_per_w // _NCHUNK
    mesh = plsc.VectorSubcoreMesh(core_axis_name="c", subcore_axis_name="s")

    @functools.partial(
        pl.kernel,
        mesh=mesh,
        out_type=jax.ShapeDtypeStruct((B, D), jnp.float32),
        scratch_types=[
            pltpu.VMEM((b_per_w,), jnp.int32),
            pltpu.VMEM((b_per_w, D), jnp.float32),
            [pltpu.SemaphoreType.DMA] * _NCHUNK,
        ],
    )
    def k(table_hbm, idx_hbm, out_hbm, idx_v, rows_v, sems):
        wid = lax.axis_index("s") * NC + lax.axis_index("c")
        base = wid * b_per_w
        pltpu.sync_copy(idx_hbm.at[pl.ds(base, b_per_w)], idx_v)
        for c in range(_NCHUNK):

            @plsc.parallel_loop(c * (CH // L), (c + 1) * (CH // L))
            def issue(j, c=c):
                v = idx_v[pl.ds(j * L, L)]
                for t in range(L):
                    pltpu.async_copy(
                        table_hbm.at[pl.ds(v[t], 1)],
                        rows_v.at[pl.ds(j * L + t, 1)],
                        sems[c],
                    )

        for c in range(_NCHUNK):
            pltpu.make_async_copy(
                table_hbm.at[pl.ds(0, CH)],
                rows_v.at[pl.ds(c * CH, CH)],
                sems[c],
            ).wait()
            pltpu.sync_copy(
                rows_v.at[pl.ds(c * CH, CH)],
                out_hbm.at[pl.ds(base + c * CH, CH)],
            )

    return k


def kernel(style_id, embed_weight):
    V, D = embed_weight.shape
    (B,) = style_id.shape
    info = plsc.get_sparse_core_info()
    idx = style_id.astype(jnp.int32)
    return _make_gather(B, D, info.num_cores, info.num_subcores)(
        embed_weight, idx
    )
